# baseline (speedup 1.0000x reference)
import functools

import jax
import jax.numpy as jnp
from jax import lax
from jax.experimental import pallas as pl
from jax.experimental.pallas import tpu as pltpu
from jax.experimental.pallas import tpu_sc as plsc

D = 64
B = 16384
NC = 2
NS = 16
NW = NC * NS
BPW = B // NW
C = 128
NCHUNK = BPW // C


def _make_kernel():
    mesh = plsc.VectorSubcoreMesh(core_axis_name="c", subcore_axis_name="s")

    @functools.partial(
        pl.kernel,
        mesh=mesh,
        out_type=jax.ShapeDtypeStruct((B,), jnp.float32),
        compiler_params=pltpu.CompilerParams(
            needs_layout_passes=False, use_tc_tiling_on_sc=False),
        scratch_types=[
            pltpu.VMEM((C,), jnp.int32),
            pltpu.VMEM((C,), jnp.int32),
            pltpu.VMEM((C,), jnp.int32),
            pltpu.VMEM((C, D), jnp.float32),
            pltpu.VMEM((C, D), jnp.float32),
            pltpu.VMEM((C, D), jnp.float32),
            pltpu.VMEM((C, D), jnp.float32),
            pltpu.VMEM((C, D), jnp.float32),
            pltpu.VMEM((C, D), jnp.float32),
            pltpu.VMEM((C * 16,), jnp.float32),
            pltpu.VMEM((BPW,), jnp.float32),
            pltpu.SemaphoreType.DMA,
        ],
    )
    def complex_score(head, relation, tail, ent_r, ent_i, rel_r, rel_i,
                      out, ih, ir, it, hr, hi, tr, ti, rr, ri, stage, out_v,
                      sem):
        wid = lax.axis_index("s") * NC + lax.axis_index("c")
        base = wid * BPW
        lane16 = lax.iota(jnp.int32, 16) * 16
        for c in range(NCHUNK):
            off = base + c * C
            pltpu.sync_copy(head.at[pl.ds(off, C)], ih)
            pltpu.sync_copy(relation.at[pl.ds(off, C)], ir)
            pltpu.sync_copy(tail.at[pl.ds(off, C)], it)
            cps = [
                pltpu.async_copy(ent_r.at[ih], hr, sem),
                pltpu.async_copy(ent_i.at[ih], hi, sem),
                pltpu.async_copy(ent_r.at[it], tr, sem),
                pltpu.async_copy(ent_i.at[it], ti, sem),
                pltpu.async_copy(rel_r.at[ir], rr, sem),
                pltpu.async_copy(rel_i.at[ir], ri, sem),
            ]
            for cp in cps:
                cp.wait()

            def row(i, _):
                acc = jnp.zeros((16,), jnp.float32)
                for k in range(D // 16):
                    s = pl.ds(k * 16, 16)
                    hrv = hr[i, s]
                    hiv = hi[i, s]
                    trv = tr[i, s]
                    tiv = ti[i, s]
                    rrv = rr[i, s]
                    riv = ri[i, s]
                    a = hrv * trv - hiv * tiv
                    b = hrv * tiv + hiv * trv
                    acc = acc + rrv * a + riv * b
                stage[pl.ds(i * 16, 16)] = acc
                return 0

            lax.fori_loop(0, C, row, 0)

            def group(g, _, c=c):
                gbase = g * 256
                tot = jnp.zeros((16,), jnp.float32)
                for j in range(16):
                    tot = tot + plsc.load_gather(stage, [gbase + lane16 + j])
                out_v[pl.ds(c * C + g * 16, 16)] = tot
                return 0

            lax.fori_loop(0, C // 16, group, 0)
        pltpu.sync_copy(out_v, out.at[pl.ds(base, BPW)])

    return complex_score


_KERNEL = _make_kernel()


def kernel(head, relation, tail, entity_real, entity_imag,
           relation_real, relation_imag):
    return _KERNEL(head, relation, tail, entity_real, entity_imag,
                   relation_real, relation_imag)
